# baseline (speedup 1.0000x reference)
import functools

import jax
import jax.numpy as jnp
from jax import lax
from jax.experimental import pallas as pl
from jax.experimental.pallas import tpu as pltpu
from jax.experimental.pallas import tpu_sc as plsc

E = 8
D = 1024
FF = 4096
EPS = 1e-5
NF = 2
FFB = FF // NF
BT = 256
TB = 16
NB = 4096 // BT + E
NP = NB * BT


NC = 2
NS = 16
NW = NC * NS
N_TOK = 4096
TPW = N_TOK // NW
CH = 64

_SC_MESH = plsc.VectorSubcoreMesh(core_axis_name="c", subcore_axis_name="s")


def _dispatch_body(feats_hbm, pos_hbm, out_hbm, idx_v, rows_v, sem):
    wid = lax.axis_index("s") * NC + lax.axis_index("c")
    base = wid * TPW
    for c in range(TPW // CH):
        off = base + c * CH
        pltpu.sync_copy(pos_hbm.at[pl.ds(off, CH)], idx_v)
        pltpu.sync_copy(feats_hbm.at[pl.ds(off, CH)], rows_v)
        pltpu.async_copy(rows_v, out_hbm.at[idx_v], sem).wait()


_dispatch = pl.kernel(
    _dispatch_body,
    out_type=jax.ShapeDtypeStruct((NP, D), jnp.float32),
    mesh=_SC_MESH,
    scratch_types=[
        pltpu.VMEM((CH,), jnp.int32),
        pltpu.VMEM((CH, D), jnp.float32),
        pltpu.SemaphoreType.DMA,
    ],
)


CH2 = 32


def _return_body(routed_hbm, pos_hbm, out_hbm, idx_v, idx2_v, r0_v, r1_v, sem):
    wid = lax.axis_index("s") * NC + lax.axis_index("c")
    base = wid * TPW
    for c in range(TPW // CH2):
        off = base + c * CH2
        pltpu.sync_copy(pos_hbm.at[pl.ds(off, CH2)], idx_v)
        for k in range(CH2 // 16):
            idx2_v[pl.ds(k * 16, 16)] = idx_v[pl.ds(k * 16, 16)] + NP
        pltpu.async_copy(routed_hbm.at[idx_v], r0_v, sem).wait()
        pltpu.async_copy(routed_hbm.at[idx2_v], r1_v, sem).wait()

        def _add_row(i, carry):
            for k in range(D // 16):
                sl = (i, pl.ds(k * 16, 16))
                r0_v[sl] = r0_v[sl] + r1_v[sl]
            return carry

        lax.fori_loop(0, CH2, _add_row, 0)
        pltpu.sync_copy(r0_v, out_hbm.at[pl.ds(off, CH2)])


_return_sum = pl.kernel(
    _return_body,
    out_type=jax.ShapeDtypeStruct((N_TOK, D), jnp.float32),
    mesh=_SC_MESH,
    scratch_types=[
        pltpu.VMEM((CH2,), jnp.int32),
        pltpu.VMEM((CH2,), jnp.int32),
        pltpu.VMEM((CH2, D), jnp.float32),
        pltpu.VMEM((CH2, D), jnp.float32),
        pltpu.SemaphoreType.DMA,
    ],
)


def _tok_map(e, f, t, nb_ref, boff_ref):
    tc = jnp.maximum(0, jnp.minimum(t, nb_ref[e] - 1))
    return (boff_ref[e] + tc, 0)


def _out_map(e, f, t, nb_ref, boff_ref):
    tc = jnp.maximum(0, jnp.minimum(t, nb_ref[e] - 1))
    return (f, boff_ref[e] + tc, 0)


def _w1_map(e, f, t, nb_ref, boff_ref):
    return (e, 0, f)


def _w2_map(e, f, t, nb_ref, boff_ref):
    return (e, f, 0)


def _b1_map(e, f, t, nb_ref, boff_ref):
    return (e, 0, f)


def _exp_map(e, f, t, nb_ref, boff_ref):
    return (e, 0, 0)


def _ffn_body(nb_ref, boff_ref, x_ref, lg_ref, lb_ref, w1_ref, b1_ref,
              w2_ref, b2_ref, o_ref):
    e = pl.program_id(0)
    f = pl.program_id(1)
    t = pl.program_id(2)

    @pl.when(t < nb_ref[e])
    def _():
        x = x_ref[...]
        mu = jnp.mean(x, axis=1, keepdims=True)
        xc = x - mu
        var = jnp.mean(xc * xc, axis=1, keepdims=True)
        h = xc * jax.lax.rsqrt(var + EPS) * lg_ref[0, 0] + lb_ref[0, 0]
        a = jnp.dot(h, w1_ref[0], preferred_element_type=jnp.float32)
        a = jnp.maximum(a + b1_ref[0, 0], 0.0)
        y = jnp.dot(a, w2_ref[0], preferred_element_type=jnp.float32)

        @pl.when(f == 0)
        def _():
            o_ref[0] = x + y + b2_ref[0, 0]

        @pl.when(f != 0)
        def _():
            o_ref[0] = y


_GRID_SPEC = pltpu.PrefetchScalarGridSpec(
    num_scalar_prefetch=2,
    grid=(E, NF, TB),
    in_specs=[
        pl.BlockSpec((BT, D), _tok_map),
        pl.BlockSpec((1, 1, D), _exp_map),
        pl.BlockSpec((1, 1, D), _exp_map),
        pl.BlockSpec((1, D, FFB), _w1_map),
        pl.BlockSpec((1, 1, FFB), _b1_map),
        pl.BlockSpec((1, FFB, D), _w2_map),
        pl.BlockSpec((1, 1, D), _exp_map),
    ],
    out_specs=pl.BlockSpec((1, BT, D), _out_map),
)


def _grouped_ffn(nb, boff, xs, lg, lb, w1e, b1e, w2e, b2e):
    return pl.pallas_call(
        _ffn_body,
        grid_spec=_GRID_SPEC,
        out_shape=jax.ShapeDtypeStruct((NF, NP, D), jnp.float32),
    )(nb, boff, xs, lg, lb, w1e, b1e, w2e, b2e)


def kernel(input_features, input_ids, hash_dict, ln_g, ln_b, w1, b1, w2, b2):
    B, S, Dm = input_features.shape
    N = B * S
    L = w1.shape[1]
    feats = input_features.reshape(N, Dm)
    ids = input_ids.reshape(N)

    tok2exp = jnp.take(hash_dict, ids, axis=0)
    onehot = (tok2exp[:, None] == jnp.arange(E, dtype=jnp.int32)[None, :])
    oh32 = onehot.astype(jnp.int32)
    csum = jnp.cumsum(oh32, axis=0)
    counts = csum[-1]
    nb = (counts + BT - 1) // BT
    boff = jnp.concatenate(
        [jnp.zeros((1,), jnp.int32), jnp.cumsum(nb)[:-1].astype(jnp.int32)])
    rank = jnp.sum(csum * oh32, axis=1) - 1
    boff_tok = jnp.sum(boff[None, :] * oh32, axis=1)
    pos = boff_tok * BT + rank

    x = feats
    for l in range(L):
        xs = _dispatch(x, pos)
        routed = _grouped_ffn(
            nb, boff, xs,
            ln_g[:, l:l + 1], ln_b[:, l:l + 1], w1[:, l], b1[:, l:l + 1],
            w2[:, l], b2[:, l:l + 1])
        x = _return_sum(routed.reshape(NF * NP, D), pos)
    return x.reshape(B, S, Dm)


# speedup vs baseline: 1.1398x; 1.1398x over previous
import functools

import jax
import jax.numpy as jnp
from jax import lax
from jax.experimental import pallas as pl
from jax.experimental.pallas import tpu as pltpu
from jax.experimental.pallas import tpu_sc as plsc

E = 8
D = 1024
FF = 4096
EPS = 1e-5
NF = 2
FFB = FF // NF
BT = 256
TB = 16
NB = 4096 // BT + E
NP = NB * BT


NC = 2
NS = 16
NW = NC * NS
N_TOK = 4096
TPW = N_TOK // NW
CH = 64

_SC_MESH = plsc.VectorSubcoreMesh(core_axis_name="c", subcore_axis_name="s")


def _dispatch_body(feats_hbm, pos_hbm, out_hbm, idx_v, rows_v, sem):
    wid = lax.axis_index("s") * NC + lax.axis_index("c")
    base = wid * TPW
    for c in range(TPW // CH):
        off = base + c * CH
        pltpu.sync_copy(pos_hbm.at[pl.ds(off, CH)], idx_v)
        pltpu.sync_copy(feats_hbm.at[pl.ds(off, CH)], rows_v)
        pltpu.async_copy(rows_v, out_hbm.at[idx_v], sem).wait()


_dispatch = pl.kernel(
    _dispatch_body,
    out_type=jax.ShapeDtypeStruct((NP, D), jnp.float32),
    mesh=_SC_MESH,
    scratch_types=[
        pltpu.VMEM((CH,), jnp.int32),
        pltpu.VMEM((CH, D), jnp.float32),
        pltpu.SemaphoreType.DMA,
    ],
)


CH2 = 32


def _return_body(routed_hbm, pos_hbm, out_hbm, idx_v, idx2_v, r0_v, r1_v, sem):
    wid = lax.axis_index("s") * NC + lax.axis_index("c")
    base = wid * TPW
    for c in range(TPW // CH2):
        off = base + c * CH2
        pltpu.sync_copy(pos_hbm.at[pl.ds(off, CH2)], idx_v)
        for k in range(CH2 // 16):
            idx2_v[pl.ds(k * 16, 16)] = idx_v[pl.ds(k * 16, 16)] + NP
        pltpu.async_copy(routed_hbm.at[idx_v], r0_v, sem).wait()
        pltpu.async_copy(routed_hbm.at[idx2_v], r1_v, sem).wait()

        def _add_row(i, carry):
            for k in range(D // 16):
                sl = (i, pl.ds(k * 16, 16))
                r0_v[sl] = r0_v[sl] + r1_v[sl]
            return carry

        lax.fori_loop(0, CH2, _add_row, 0)
        pltpu.sync_copy(r0_v, out_hbm.at[pl.ds(off, CH2)])


_return_sum = pl.kernel(
    _return_body,
    out_type=jax.ShapeDtypeStruct((N_TOK, D), jnp.float32),
    mesh=_SC_MESH,
    scratch_types=[
        pltpu.VMEM((CH2,), jnp.int32),
        pltpu.VMEM((CH2,), jnp.int32),
        pltpu.VMEM((CH2, D), jnp.float32),
        pltpu.VMEM((CH2, D), jnp.float32),
        pltpu.SemaphoreType.DMA,
    ],
)


def _tok_map(f, b, eb_ref):
    return (b, 0)


def _out_map(f, b, eb_ref):
    return (f, b, 0)


def _w1_map(f, b, eb_ref):
    return (eb_ref[b], 0, f)


def _w2_map(f, b, eb_ref):
    return (eb_ref[b], f, 0)


def _b1_map(f, b, eb_ref):
    return (eb_ref[b], 0, f)


def _exp_map(f, b, eb_ref):
    return (eb_ref[b], 0, 0)


def _ffn_body(eb_ref, x_ref, lg_ref, lb_ref, w1_ref, b1_ref,
              w2_ref, b2_ref, o_ref):
    f = pl.program_id(0)
    x = x_ref[...]
    mu = jnp.mean(x, axis=1, keepdims=True)
    xc = x - mu
    var = jnp.mean(xc * xc, axis=1, keepdims=True)
    h = xc * jax.lax.rsqrt(var + EPS) * lg_ref[0, 0] + lb_ref[0, 0]
    a = jnp.dot(h, w1_ref[0], preferred_element_type=jnp.float32)
    a = jnp.maximum(a + b1_ref[0, 0], 0.0)
    y = jnp.dot(a, w2_ref[0], preferred_element_type=jnp.float32)

    @pl.when(f == 0)
    def _():
        o_ref[0] = x + y + b2_ref[0, 0]

    @pl.when(f != 0)
    def _():
        o_ref[0] = y


_GRID_SPEC = pltpu.PrefetchScalarGridSpec(
    num_scalar_prefetch=1,
    grid=(NF, NB),
    in_specs=[
        pl.BlockSpec((BT, D), _tok_map),
        pl.BlockSpec((1, 1, D), _exp_map),
        pl.BlockSpec((1, 1, D), _exp_map),
        pl.BlockSpec((1, D, FFB), _w1_map),
        pl.BlockSpec((1, 1, FFB), _b1_map),
        pl.BlockSpec((1, FFB, D), _w2_map),
        pl.BlockSpec((1, 1, D), _exp_map),
    ],
    out_specs=pl.BlockSpec((1, BT, D), _out_map),
)


def _grouped_ffn(eb, xs, lg, lb, w1e, b1e, w2e, b2e):
    return pl.pallas_call(
        _ffn_body,
        grid_spec=_GRID_SPEC,
        out_shape=jax.ShapeDtypeStruct((NF, NP, D), jnp.float32),
    )(eb, xs, lg, lb, w1e, b1e, w2e, b2e)


def kernel(input_features, input_ids, hash_dict, ln_g, ln_b, w1, b1, w2, b2):
    B, S, Dm = input_features.shape
    N = B * S
    L = w1.shape[1]
    feats = input_features.reshape(N, Dm)
    ids = input_ids.reshape(N)

    tok2exp = jnp.take(hash_dict, ids, axis=0)
    onehot = (tok2exp[:, None] == jnp.arange(E, dtype=jnp.int32)[None, :])
    oh32 = onehot.astype(jnp.int32)
    csum = jnp.cumsum(oh32, axis=0)
    counts = csum[-1]
    nb = (counts + BT - 1) // BT
    boff = jnp.concatenate(
        [jnp.zeros((1,), jnp.int32), jnp.cumsum(nb)[:-1].astype(jnp.int32)])
    rank = jnp.sum(csum * oh32, axis=1) - 1
    boff_tok = jnp.sum(boff[None, :] * oh32, axis=1)
    pos = boff_tok * BT + rank
    eb = (jnp.sum(jnp.arange(NB, dtype=jnp.int32)[:, None] >=
                  boff[None, :], axis=1) - 1).astype(jnp.int32)

    x = feats
    for l in range(L):
        xs = _dispatch(x, pos)
        routed = _grouped_ffn(
            eb, xs,
            ln_g[:, l:l + 1], ln_b[:, l:l + 1], w1[:, l], b1[:, l:l + 1],
            w2[:, l], b2[:, l:l + 1])
        x = _return_sum(routed.reshape(NF * NP, D), pos)
    return x.reshape(B, S, Dm)
